# baseline (speedup 1.0000x reference)
import jax
import jax.numpy as jnp
from jax.experimental import pallas as pl

_ROWS = 1024


def _bcast_copy(pos_ref, out_ref):
    out_ref[...] = pos_ref[...][None]


def kernel(x, pos_embedding):
    B, S = x.shape
    H = pos_embedding.shape[1]
    grid = (S // _ROWS, B)
    return pl.pallas_call(
        _bcast_copy,
        grid=grid,
        in_specs=[pl.BlockSpec((_ROWS, H), lambda i, b: (i, 0))],
        out_specs=pl.BlockSpec((1, _ROWS, H), lambda i, b: (b, i, 0)),
        out_shape=jax.ShapeDtypeStruct((B, S, H), pos_embedding.dtype),
    )(pos_embedding)


# speedup vs baseline: 1.2799x; 1.2799x over previous
import jax
import jax.numpy as jnp
from jax.experimental import pallas as pl
from jax.experimental.pallas import tpu as pltpu

_CHUNK = 1024


def _fanout_body(pos_hbm, out_hbm, vmem, in_sem, out_sem):
    n = pos_hbm.shape[0] // _CHUNK
    batch = out_hbm.shape[0]

    def in_copy(c):
        return pltpu.make_async_copy(
            pos_hbm.at[pl.ds(c * _CHUNK, _CHUNK), :], vmem.at[c], in_sem.at[c]
        )

    def out_copy(c, b):
        return pltpu.make_async_copy(
            vmem.at[c], out_hbm.at[b, pl.ds(c * _CHUNK, _CHUNK), :], out_sem.at[c]
        )

    for c in range(n):
        in_copy(c).start()
    for c in range(n):
        in_copy(c).wait()
        for b in range(batch):
            out_copy(c, b).start()
    for c in range(n):
        for b in range(batch):
            out_copy(c, b).wait()


def kernel(x, pos_embedding):
    B, S = x.shape
    H = pos_embedding.shape[1]
    n = S // _CHUNK
    return pl.pallas_call(
        _fanout_body,
        in_specs=[pl.BlockSpec(memory_space=pl.ANY)],
        out_specs=pl.BlockSpec(memory_space=pl.ANY),
        out_shape=jax.ShapeDtypeStruct((B, S, H), pos_embedding.dtype),
        scratch_shapes=[
            pltpu.VMEM((n, _CHUNK, H), pos_embedding.dtype),
            pltpu.SemaphoreType.DMA((n,)),
            pltpu.SemaphoreType.DMA((n,)),
        ],
    )(pos_embedding)


# speedup vs baseline: 1.2803x; 1.0003x over previous
import jax
import jax.numpy as jnp
from jax.experimental import pallas as pl
from jax.experimental.pallas import tpu as pltpu

_CHUNK = 512


def _fanout_body(pos_hbm, out_hbm, vmem, in_sem, out_sem):
    n = pos_hbm.shape[0] // _CHUNK
    batch = out_hbm.shape[0]

    def in_copy(c):
        return pltpu.make_async_copy(
            pos_hbm.at[pl.ds(c * _CHUNK, _CHUNK), :], vmem.at[c], in_sem.at[c]
        )

    def out_copy(c, b):
        return pltpu.make_async_copy(
            vmem.at[c], out_hbm.at[b, pl.ds(c * _CHUNK, _CHUNK), :], out_sem.at[c]
        )

    for c in range(n):
        in_copy(c).start()
    for c in range(n):
        in_copy(c).wait()
        for b in range(batch):
            out_copy(c, b).start()
    for c in range(n):
        for b in range(batch):
            out_copy(c, b).wait()


def kernel(x, pos_embedding):
    B, S = x.shape
    H = pos_embedding.shape[1]
    n = S // _CHUNK
    return pl.pallas_call(
        _fanout_body,
        in_specs=[pl.BlockSpec(memory_space=pl.ANY)],
        out_specs=pl.BlockSpec(memory_space=pl.ANY),
        out_shape=jax.ShapeDtypeStruct((B, S, H), pos_embedding.dtype),
        scratch_shapes=[
            pltpu.VMEM((n, _CHUNK, H), pos_embedding.dtype),
            pltpu.SemaphoreType.DMA((n,)),
            pltpu.SemaphoreType.DMA((n,)),
        ],
    )(pos_embedding)
